# baseline (speedup 1.0000x reference)
import jax
import jax.numpy as jnp
from jax.experimental import pallas as pl
from jax.experimental.pallas import tpu as pltpu

MAXLEN = 200
ZBLOCK = 1


def _fill_body(x_ref, o_ref, zbuf, sem):
    zbuf[...] = jnp.zeros_like(zbuf)
    nfull = (MAXLEN - 1) // ZBLOCK
    tail = MAXLEN - 1 - nfull * ZBLOCK
    copies = []
    for i in range(nfull):
        copies.append(
            pltpu.make_async_copy(zbuf, o_ref.at[pl.ds(i * ZBLOCK, ZBLOCK)], sem)
        )
    if tail:
        copies.append(
            pltpu.make_async_copy(
                zbuf.at[pl.ds(0, tail)], o_ref.at[pl.ds(nfull * ZBLOCK, tail)], sem
            )
        )
    copies.insert(0, pltpu.make_async_copy(x_ref, o_ref.at[pl.ds(MAXLEN - 1, 1)], sem))
    for c in copies:
        c.start()
    for c in copies:
        c.wait()


def kernel(input):
    n, d = input.shape
    return pl.pallas_call(
        _fill_body,
        in_specs=[pl.BlockSpec(memory_space=pl.ANY)],
        out_specs=pl.BlockSpec(memory_space=pl.ANY),
        out_shape=jax.ShapeDtypeStruct((MAXLEN, n, d), input.dtype),
        scratch_shapes=[
            pltpu.VMEM((ZBLOCK, n, d), input.dtype),
            pltpu.SemaphoreType.DMA,
        ],
    )(input.reshape(1, n, d))
